# Initial kernel scaffold; baseline (speedup 1.0000x reference)
import jax
import jax.numpy as jnp
from jax.experimental import pallas as pl


def kernel(hidden_states, cos, sin, Wq_a, q_norm_w, Wq_b, Wkv, Wo, Wiq, Wik):
    raise NotImplementedError("write your pallas kernel here")


# speedup vs baseline: 10.5675x; 10.5675x over previous
import functools

import jax
import jax.numpy as jnp
import numpy as np
from jax import lax
from jax.experimental import pallas as pl
from jax.experimental.pallas import tpu as pltpu

_N_HEADS = 16
_HEAD_DIM = 192
_ROPE_DIM = 64
_NOPE_DIM = _HEAD_DIM - _ROPE_DIM
_IDX_DIM = 128
_TOPK = 512
_EPS = 1e-6
_BQ = 256


def _proj_body(hs_ref, wqa_ref, qnw_ref, wqb_ref, wkv_ref, wiq_ref, wik_ref,
               cq_ref, sq_ref, ck_ref, sk_ref, pq_ref, pk_ref,
               qn_out, qr_out, kn_out, kr_out, iq_out, ik_out, *, n_heads):
    x = hs_ref[...]
    xb = x.astype(jnp.bfloat16)
    f32 = jnp.float32
    qa = lax.dot_general(xb, wqa_ref[...], (((1,), (0,)), ((), ())),
                         preferred_element_type=f32)
    qa = qa * lax.rsqrt(jnp.mean(qa * qa, axis=-1, keepdims=True) + _EPS)
    qa = qa * qnw_ref[...]
    q = lax.dot_general(qa.astype(jnp.bfloat16), wqb_ref[...],
                        (((1,), (0,)), ((), ())), preferred_element_type=f32)
    nh_nope = n_heads * _NOPE_DIM
    qn = q[:, :nh_nope]
    qr = q[:, nh_nope:]
    qr_sw = lax.dot_general(qr, pq_ref[...], (((1,), (0,)), ((), ())),
                            preferred_element_type=f32)
    qr = qr * cq_ref[...] + qr_sw * sq_ref[...]
    qn_out[...] = qn.astype(jnp.bfloat16)
    qr_out[...] = qr.astype(jnp.bfloat16)
    kv = lax.dot_general(xb, wkv_ref[...], (((1,), (0,)), ((), ())),
                         preferred_element_type=f32)
    kn = kv[:, :_NOPE_DIM]
    kr = kv[:, _NOPE_DIM:]
    kr_sw = lax.dot_general(kr, pk_ref[...], (((1,), (0,)), ((), ())),
                            preferred_element_type=f32)
    kr = kr * ck_ref[...] + kr_sw * sk_ref[...]
    kn_out[...] = kn.astype(jnp.bfloat16)
    kr_out[...] = kr.astype(jnp.bfloat16)
    iq_out[...] = lax.dot_general(x, wiq_ref[...], (((1,), (0,)), ((), ())),
                                  preferred_element_type=f32)
    ik_out[...] = lax.dot_general(x, wik_ref[...], (((1,), (0,)), ((), ())),
                                  preferred_element_type=f32)


def _attn_body(qn_ref, qr_ref, iq_ref, kn_ref, kr_ref, ik_ref, wo_ref, out_ref,
               *, bq, seq, topk, n_heads):
    i32 = jnp.int32
    f32 = jnp.float32
    blk = pl.program_id(0)
    rows = blk * bq + lax.broadcasted_iota(i32, (bq, seq), 0)
    cols = lax.broadcasted_iota(i32, (bq, seq), 1)
    causal = cols <= rows
    sidx = lax.dot_general(iq_ref[...], ik_ref[...], (((1,), (1,)), ((), ())),
                           preferred_element_type=f32) * (_IDX_DIM ** -0.5)
    sidx = jnp.where(causal, sidx, -jnp.inf)
    ti = lax.bitcast_convert_type(sidx, i32)
    ti = ti ^ ((ti >> 31) & jnp.int32(0x7FFFFFFF))
    kk = jnp.minimum(jnp.int32(topk), rows[:, :1] + 1)
    lo = jnp.full((bq, 1), jnp.iinfo(jnp.int32).min, i32)
    hi = jnp.full((bq, 1), jnp.iinfo(jnp.int32).max, i32)
    for _ in range(32):
        mid = (lo & hi) + ((lo ^ hi) >> 1)
        cnt = jnp.sum((ti >= mid).astype(i32), axis=1, keepdims=True)
        pred = cnt >= kk
        lo = jnp.where(pred, mid, lo)
        hi = jnp.where(pred, hi, mid)
    keep = ti >= lo
    scaling = _HEAD_DIM ** -0.5
    acc = jnp.zeros((bq, out_ref.shape[-1]), f32)
    for h in range(n_heads):
        qn_h = qn_ref[:, h * _NOPE_DIM:(h + 1) * _NOPE_DIM]
        qr_h = qr_ref[:, h * _ROPE_DIM:(h + 1) * _ROPE_DIM]
        s = lax.dot_general(qn_h, kn_ref[...], (((1,), (1,)), ((), ())),
                            preferred_element_type=f32)
        s += lax.dot_general(qr_h, kr_ref[...], (((1,), (1,)), ((), ())),
                             preferred_element_type=f32)
        s = jnp.where(keep, s * scaling, -1e30)
        m = jnp.max(s, axis=1, keepdims=True)
        p = jnp.exp(s - m)
        p = (p / jnp.sum(p, axis=1, keepdims=True)).astype(jnp.bfloat16)
        o_n = lax.dot_general(p, kn_ref[...], (((1,), (0,)), ((), ())),
                              preferred_element_type=f32)
        o_r = lax.dot_general(p, kr_ref[...], (((1,), (0,)), ((), ())),
                              preferred_element_type=f32)
        wo_n = wo_ref[h * _HEAD_DIM:h * _HEAD_DIM + _NOPE_DIM, :]
        wo_r = wo_ref[h * _HEAD_DIM + _NOPE_DIM:(h + 1) * _HEAD_DIM, :]
        acc += lax.dot_general(o_n.astype(jnp.bfloat16), wo_n,
                               (((1,), (0,)), ((), ())),
                               preferred_element_type=f32)
        acc += lax.dot_general(o_r.astype(jnp.bfloat16), wo_r,
                               (((1,), (0,)), ((), ())),
                               preferred_element_type=f32)
    out_ref[...] = acc


def _impl(hidden_states, cos, sin, Wq_a, q_norm_w, Wq_b, Wkv, Wo, Wiq, Wik,
          topk=_TOPK):
    hs = hidden_states[0]
    seq, hdim = hs.shape
    n_heads = Wq_b.shape[1] // _HEAD_DIM
    nh_nope = n_heads * _NOPE_DIM
    nh_rope = n_heads * _ROPE_DIM
    bq = min(_BQ, seq)
    grid = seq // bq
    bf16 = jnp.bfloat16
    f32 = jnp.float32

    cos2 = jnp.repeat(cos, 2, axis=-1)
    sin2 = jnp.stack([-sin, sin], axis=-1).reshape(seq, _ROPE_DIM)
    cq = jnp.tile(cos2, (1, n_heads))
    sq = jnp.tile(sin2, (1, n_heads))
    perm64 = np.arange(_ROPE_DIM) ^ 1
    p64 = jnp.asarray(np.eye(_ROPE_DIM, dtype=np.float32)[perm64])
    pq = jnp.asarray(np.kron(np.eye(n_heads, dtype=np.float32),
                             np.eye(_ROPE_DIM, dtype=np.float32)[perm64]))
    col = np.arange(n_heads * _HEAD_DIM).reshape(n_heads, _HEAD_DIM)
    perm_cols = np.concatenate([col[:, :_NOPE_DIM].reshape(-1),
                                col[:, _NOPE_DIM:].reshape(-1)])
    wqb_p = jnp.take(Wq_b, jnp.asarray(perm_cols), axis=1).astype(bf16)

    proj = pl.pallas_call(
        functools.partial(_proj_body, n_heads=n_heads),
        grid=(grid,),
        in_specs=[
            pl.BlockSpec((bq, hdim), lambda i: (i, 0)),
            pl.BlockSpec((hdim, Wq_a.shape[1]), lambda i: (0, 0)),
            pl.BlockSpec((1, q_norm_w.shape[0]), lambda i: (0, 0)),
            pl.BlockSpec((Wq_b.shape[0], Wq_b.shape[1]), lambda i: (0, 0)),
            pl.BlockSpec((hdim, _HEAD_DIM), lambda i: (0, 0)),
            pl.BlockSpec((hdim, _IDX_DIM), lambda i: (0, 0)),
            pl.BlockSpec((hdim, _IDX_DIM), lambda i: (0, 0)),
            pl.BlockSpec((bq, nh_rope), lambda i: (i, 0)),
            pl.BlockSpec((bq, nh_rope), lambda i: (i, 0)),
            pl.BlockSpec((bq, _ROPE_DIM), lambda i: (i, 0)),
            pl.BlockSpec((bq, _ROPE_DIM), lambda i: (i, 0)),
            pl.BlockSpec((nh_rope, nh_rope), lambda i: (0, 0)),
            pl.BlockSpec((_ROPE_DIM, _ROPE_DIM), lambda i: (0, 0)),
        ],
        out_specs=[
            pl.BlockSpec((bq, nh_nope), lambda i: (i, 0)),
            pl.BlockSpec((bq, nh_rope), lambda i: (i, 0)),
            pl.BlockSpec((bq, _NOPE_DIM), lambda i: (i, 0)),
            pl.BlockSpec((bq, _ROPE_DIM), lambda i: (i, 0)),
            pl.BlockSpec((bq, _IDX_DIM), lambda i: (i, 0)),
            pl.BlockSpec((bq, _IDX_DIM), lambda i: (i, 0)),
        ],
        out_shape=[
            jax.ShapeDtypeStruct((seq, nh_nope), bf16),
            jax.ShapeDtypeStruct((seq, nh_rope), bf16),
            jax.ShapeDtypeStruct((seq, _NOPE_DIM), bf16),
            jax.ShapeDtypeStruct((seq, _ROPE_DIM), bf16),
            jax.ShapeDtypeStruct((seq, _IDX_DIM), f32),
            jax.ShapeDtypeStruct((seq, _IDX_DIM), f32),
        ],
    )
    qn, qr, kn, kr, iq, ik = proj(
        hs, Wq_a.astype(bf16), q_norm_w.reshape(1, -1), wqb_p,
        Wkv.astype(bf16), Wiq, Wik, cq, sq, cos2, sin2, pq, p64)

    attn = pl.pallas_call(
        functools.partial(_attn_body, bq=bq, seq=seq, topk=topk,
                          n_heads=n_heads),
        grid=(grid,),
        in_specs=[
            pl.BlockSpec((bq, nh_nope), lambda i: (i, 0)),
            pl.BlockSpec((bq, nh_rope), lambda i: (i, 0)),
            pl.BlockSpec((bq, _IDX_DIM), lambda i: (i, 0)),
            pl.BlockSpec((seq, _NOPE_DIM), lambda i: (0, 0)),
            pl.BlockSpec((seq, _ROPE_DIM), lambda i: (0, 0)),
            pl.BlockSpec((seq, _IDX_DIM), lambda i: (0, 0)),
            pl.BlockSpec((n_heads * _HEAD_DIM, hdim), lambda i: (0, 0)),
        ],
        out_specs=pl.BlockSpec((bq, hdim), lambda i: (i, 0)),
        out_shape=jax.ShapeDtypeStruct((seq, hdim), f32),
    )
    out = attn(qn, qr, iq, kn, kr, ik, Wo.astype(bf16))
    return out[None]


def kernel(hidden_states, cos, sin, Wq_a, q_norm_w, Wq_b, Wkv, Wo, Wiq, Wik):
    return _impl(hidden_states, cos, sin, Wq_a, q_norm_w, Wq_b, Wkv, Wo,
                 Wiq, Wik)
